# baseline (speedup 1.0000x reference)
import functools
import math

import numpy as np
import jax
import jax.numpy as jnp
from jax import lax
from jax.experimental import pallas as pl
from jax.experimental.pallas import tpu as pltpu
from jax.experimental.pallas import tpu_sc as plsc

_D = 1024
_SCALE = math.sqrt(_D)
_NC, _NS = 2, 16
_NW = _NC * _NS
_K = 16
_LANES = 16
_CPR = _D // _LANES
_UNROLL = 4


def _make_pe(seq, d):
    position = np.arange(0, seq, dtype=np.float32)[:, None]
    div_term = np.exp(
        np.arange(0, d, 2, dtype=np.float32) * (-math.log(10000.0) / d))
    pe = np.zeros((seq, d), dtype=np.float32)
    pe[:, 0::2] = np.sin(position * div_term)
    pe[:, 1::2] = np.cos(position * div_term)
    return pe


@functools.partial(jax.jit, static_argnames=("nb", "seq"))
def _sc_embed(x_flat, table, pe, nb, seq):
    ppw = seq // _NW
    nwin = ppw // _K
    nchunk = nwin * nb
    assert nwin >= 3 and nb == 4
    mesh = plsc.VectorSubcoreMesh(core_axis_name="c", subcore_axis_name="s")

    @functools.partial(
        pl.kernel,
        out_type=jax.ShapeDtypeStruct((nb * seq, _D), jnp.float32),
        mesh=mesh,
        scratch_types=[
            pltpu.VMEM((nb * ppw,), jnp.int32),
            pltpu.VMEM((_K, _D), jnp.float32),
            pltpu.VMEM((_K, _D), jnp.float32),
            pltpu.VMEM((_K, _D), jnp.float32),
            pltpu.VMEM((_K, _D), jnp.float32),
            pltpu.VMEM((_K, _D), jnp.float32),
            pltpu.VMEM((_K, _D), jnp.float32),
            [pltpu.SemaphoreType.DMA] * 10,
        ],
    )
    def k(x_hbm, table_hbm, pe_hbm, out_hbm, idx_v, rows0, rows1, rows2,
          rows3, pe0, pe1, sems):
        g0, g1, g2, g3, o0, o1, o2, o3, p0, p1 = sems
        wid = lax.axis_index("s") * _NC + lax.axis_index("c")
        pbase = wid * ppw

        rows = [rows0, rows1, rows2, rows3]
        pev = [pe0, pe1]
        gsem = [g0, g1, g2, g3]
        osem = [o0, o1, o2, o3]
        psem = [p0, p1]

        idx_descs = [
            pltpu.make_async_copy(
                x_hbm.at[pl.ds(b * seq + pbase, ppw)],
                idx_v.at[pl.ds(b * ppw, ppw)],
                [g1, g2, g3, p1][b])
            for b in range(nb)
        ]
        for d in idx_descs:
            d.start()
        for d in idx_descs:
            d.wait()

        def gather_desc(v, b, r4):
            return pltpu.make_async_copy(
                table_hbm.at[idx_v.at[pl.ds(b * ppw + v * _K, _K)]],
                rows[r4], gsem[r4])

        def pe_desc(v, p2):
            return pltpu.make_async_copy(
                pe_hbm.at[pl.ds(pbase + v * _K, _K)], pev[p2], psem[p2])

        def out_desc(v, b, r4):
            return pltpu.make_async_copy(
                rows[r4],
                out_hbm.at[pl.ds(b * seq + pbase + v * _K, _K)], osem[r4])

        def compute(r4, p2):
            rows_b = rows[r4]
            pe_b = pev[p2]

            def compute_body(it, carry):
                i0 = it * _UNROLL
                for u in range(_UNROLL):
                    i = i0 + u
                    r = lax.shift_right_logical(i, 6)
                    j = lax.bitwise_and(i, _CPR - 1)
                    sl = pl.ds(j * _LANES, _LANES)
                    rows_b[r, sl] = rows_b[r, sl] * _SCALE + pe_b[r, sl]
                return carry

            lax.fori_loop(0, _K * _CPR // _UNROLL, compute_body, 0)

        def chunk_step(v, b, wait_out, start_next, p2, wait_pe):
            r4 = b
            if wait_out:
                b3 = b - 3
                v3, bb3 = (v, b3) if b3 >= 0 else (v - 1, b3 + nb)
                out_desc(v3, bb3, (b + 1) % 4).wait()
            if start_next:
                bn = b + 1
                vn, bbn = (v, bn) if bn < nb else (v + 1, 0)
                gather_desc(vn, bbn, (b + 1) % 4).start()
            gather_desc(v, b, r4).wait()
            if wait_pe:
                pe_desc(v, p2).wait()
            compute(r4, p2)
            out_desc(v, b, r4).start()

        def do_window(v, p2, first_win, last_win):
            if not last_win:
                pe_desc(v + 1, p2 ^ 1).start()
            for b in range(nb):
                wait_out = not (first_win and b < 3)
                start_next = not (last_win and b == nb - 1)
                chunk_step(v, b, wait_out, start_next, p2, b == 0)

        gather_desc(0, 0, 0).start()
        pe_desc(0, 0).start()
        do_window(0, 0, True, False)

        def pair_body(t, carry):
            v = 1 + 2 * t
            do_window(v, 1, False, False)
            do_window(v + 1, 0, False, False)
            return carry

        lax.fori_loop(0, (nwin - 2) // 2, pair_body, 0)

        do_window(nwin - 1, (nwin - 1) & 1, False, True)
        for b in range(1, nb):
            out_desc(nwin - 1, b, b).wait()

    return k(x_flat, table, pe)


def kernel(x, table):
    b, s = x.shape
    pe = jnp.asarray(_make_pe(s, _D))
    x_flat = x.reshape(-1).astype(jnp.int32)
    out = _sc_embed(x_flat, table, pe, b, s)
    return out.reshape(b, s, _D)


# speedup vs baseline: 1.0259x; 1.0259x over previous
import functools
import math

import numpy as np
import jax
import jax.numpy as jnp
from jax import lax
from jax.experimental import pallas as pl
from jax.experimental.pallas import tpu as pltpu
from jax.experimental.pallas import tpu_sc as plsc

_D = 1024
_SCALE = math.sqrt(_D)
_NC, _NS = 2, 16
_NW = _NC * _NS
_K = 16
_LANES = 16
_CPR = _D // _LANES
_UNROLL = 8


def _make_pe(seq, d):
    position = np.arange(0, seq, dtype=np.float32)[:, None]
    div_term = np.exp(
        np.arange(0, d, 2, dtype=np.float32) * (-math.log(10000.0) / d))
    pe = np.zeros((seq, d), dtype=np.float32)
    pe[:, 0::2] = np.sin(position * div_term)
    pe[:, 1::2] = np.cos(position * div_term)
    return pe


@functools.partial(jax.jit, static_argnames=("nb", "seq"))
def _sc_embed(x_flat, table, pe, nb, seq):
    ppw = seq // _NW
    nwin = ppw // _K
    nchunk = nwin * nb
    assert nwin >= 3 and nb == 4
    mesh = plsc.VectorSubcoreMesh(core_axis_name="c", subcore_axis_name="s")

    @functools.partial(
        pl.kernel,
        out_type=jax.ShapeDtypeStruct((nb * seq, _D), jnp.float32),
        mesh=mesh,
        scratch_types=[
            pltpu.VMEM((nb * ppw,), jnp.int32),
            pltpu.VMEM((_K, _D), jnp.float32),
            pltpu.VMEM((_K, _D), jnp.float32),
            pltpu.VMEM((_K, _D), jnp.float32),
            pltpu.VMEM((_K, _D), jnp.float32),
            pltpu.VMEM((_K, _D), jnp.float32),
            pltpu.VMEM((_K, _D), jnp.float32),
            [pltpu.SemaphoreType.DMA] * 10,
        ],
    )
    def k(x_hbm, table_hbm, pe_hbm, out_hbm, idx_v, rows0, rows1, rows2,
          rows3, pe0, pe1, sems):
        g0, g1, g2, g3, o0, o1, o2, o3, p0, p1 = sems
        wid = lax.axis_index("s") * _NC + lax.axis_index("c")
        pbase = wid * ppw

        rows = [rows0, rows1, rows2, rows3]
        pev = [pe0, pe1]
        gsem = [g0, g1, g2, g3]
        osem = [o0, o1, o2, o3]
        psem = [p0, p1]

        idx_descs = [
            pltpu.make_async_copy(
                x_hbm.at[pl.ds(b * seq + pbase, ppw)],
                idx_v.at[pl.ds(b * ppw, ppw)],
                [g1, g2, g3, p1][b])
            for b in range(nb)
        ]
        for d in idx_descs:
            d.start()
        for d in idx_descs:
            d.wait()

        def gather_desc(v, b, r4):
            return pltpu.make_async_copy(
                table_hbm.at[idx_v.at[pl.ds(b * ppw + v * _K, _K)]],
                rows[r4], gsem[r4])

        def pe_desc(v, p2):
            return pltpu.make_async_copy(
                pe_hbm.at[pl.ds(pbase + v * _K, _K)], pev[p2], psem[p2])

        def out_desc(v, b, r4):
            return pltpu.make_async_copy(
                rows[r4],
                out_hbm.at[pl.ds(b * seq + pbase + v * _K, _K)], osem[r4])

        def compute(r4, p2):
            rows_b = rows[r4]
            pe_b = pev[p2]

            def compute_body(it, carry):
                i0 = it * _UNROLL
                for u in range(_UNROLL):
                    i = i0 + u
                    r = lax.shift_right_logical(i, 6)
                    j = lax.bitwise_and(i, _CPR - 1)
                    sl = pl.ds(j * _LANES, _LANES)
                    rows_b[r, sl] = rows_b[r, sl] * _SCALE + pe_b[r, sl]
                return carry

            lax.fori_loop(0, _K * _CPR // _UNROLL, compute_body, 0)

        def chunk_step(v, b, wait_out, start_next, p2, wait_pe):
            r4 = b
            if wait_out:
                b3 = b - 3
                v3, bb3 = (v, b3) if b3 >= 0 else (v - 1, b3 + nb)
                out_desc(v3, bb3, (b + 1) % 4).wait()
            if start_next:
                bn = b + 1
                vn, bbn = (v, bn) if bn < nb else (v + 1, 0)
                gather_desc(vn, bbn, (b + 1) % 4).start()
            gather_desc(v, b, r4).wait()
            if wait_pe:
                pe_desc(v, p2).wait()
            compute(r4, p2)
            out_desc(v, b, r4).start()

        def do_window(v, p2, first_win, last_win):
            if not last_win:
                pe_desc(v + 1, p2 ^ 1).start()
            for b in range(nb):
                wait_out = not (first_win and b < 3)
                start_next = not (last_win and b == nb - 1)
                chunk_step(v, b, wait_out, start_next, p2, b == 0)

        gather_desc(0, 0, 0).start()
        pe_desc(0, 0).start()
        do_window(0, 0, True, False)

        def pair_body(t, carry):
            v = 1 + 2 * t
            do_window(v, 1, False, False)
            do_window(v + 1, 0, False, False)
            return carry

        lax.fori_loop(0, (nwin - 2) // 2, pair_body, 0)

        do_window(nwin - 1, (nwin - 1) & 1, False, True)
        for b in range(1, nb):
            out_desc(nwin - 1, b, b).wait()

    return k(x_flat, table, pe)


def kernel(x, table):
    b, s = x.shape
    pe = jnp.asarray(_make_pe(s, _D))
    x_flat = x.reshape(-1).astype(jnp.int32)
    out = _sc_embed(x_flat, table, pe, b, s)
    return out.reshape(b, s, _D)


# speedup vs baseline: 1.0311x; 1.0050x over previous
import functools
import math

import numpy as np
import jax
import jax.numpy as jnp
from jax import lax
from jax.experimental import pallas as pl
from jax.experimental.pallas import tpu as pltpu
from jax.experimental.pallas import tpu_sc as plsc

_D = 1024
_SCALE = math.sqrt(_D)
_NC, _NS = 2, 16
_NW = _NC * _NS
_K = 16
_LANES = 16
_CPR = _D // _LANES
_UNROLL = 8


def _make_pe(seq, d):
    position = np.arange(0, seq, dtype=np.float32)[:, None]
    div_term = np.exp(
        np.arange(0, d, 2, dtype=np.float32) * (-math.log(10000.0) / d))
    pe = np.zeros((seq, d), dtype=np.float32)
    pe[:, 0::2] = np.sin(position * div_term)
    pe[:, 1::2] = np.cos(position * div_term)
    return pe


@functools.partial(jax.jit, static_argnames=("nb", "seq"))
def _sc_embed(x_flat, table, pe, nb, seq):
    ppw = seq // _NW
    nwin = ppw // _K
    assert nwin >= 4 and nwin % 2 == 0 and nb == 4 and seq % _NW == 0
    mesh = plsc.VectorSubcoreMesh(core_axis_name="c", subcore_axis_name="s")

    @functools.partial(
        pl.kernel,
        out_type=jax.ShapeDtypeStruct((nb * seq, _D), jnp.float32),
        mesh=mesh,
        scratch_types=[
            pltpu.VMEM((nb * ppw,), jnp.int32),
            pltpu.VMEM((_K, _D), jnp.float32),
            pltpu.VMEM((_K, _D), jnp.float32),
            pltpu.VMEM((_K, _D), jnp.float32),
            pltpu.VMEM((_K, _D), jnp.float32),
            pltpu.VMEM((_K, _D), jnp.float32),
            pltpu.VMEM((_K, _D), jnp.float32),
            [pltpu.SemaphoreType.DMA] * 10,
        ],
    )
    def k(x_hbm, table_hbm, pe_hbm, out_hbm, idx_v, rows0, rows1, rows2,
          rows3, pe0, pe1, sems):
        g0, g1, g2, g3, o0, o1, o2, o3, p0, p1 = sems
        wid = lax.axis_index("s") * _NC + lax.axis_index("c")
        pbase = wid * ppw

        rows = [rows0, rows1, rows2, rows3]
        pev = [pe0, pe1]
        gsem = [g0, g1, g2, g3]
        osem = [o0, o1, o2, o3]
        psem = [p0, p1]

        idx_descs = [
            pltpu.make_async_copy(
                x_hbm.at[pl.ds(b * seq + pbase, ppw)],
                idx_v.at[pl.ds(b * ppw, ppw)],
                [g1, g2, g3, p1][b])
            for b in range(nb)
        ]
        for d in idx_descs:
            d.start()
        for d in idx_descs:
            d.wait()

        def gather_desc(v, b, r4):
            return pltpu.make_async_copy(
                table_hbm.at[idx_v.at[pl.ds(b * ppw + v * _K, _K)]],
                rows[r4], gsem[r4])

        def pe_desc(v, p2):
            return pltpu.make_async_copy(
                pe_hbm.at[pl.ds(pbase + v * _K, _K)], pev[p2], psem[p2])

        def out_desc(v, b, r4):
            return pltpu.make_async_copy(
                rows[r4],
                out_hbm.at[pl.ds(b * seq + pbase + v * _K, _K)], osem[r4])

        def compute(r4, p2):
            rows_b = rows[r4]
            pe_b = pev[p2]

            def compute_body(it, carry):
                i0 = it * _UNROLL
                for u in range(_UNROLL):
                    i = i0 + u
                    r = lax.shift_right_logical(i, 6)
                    j = lax.bitwise_and(i, _CPR - 1)
                    sl = pl.ds(j * _LANES, _LANES)
                    rows_b[r, sl] = rows_b[r, sl] * _SCALE + pe_b[r, sl]
                return carry

            lax.fori_loop(0, _K * _CPR // _UNROLL, compute_body, 0)

        def chunk_step(v, b, wait_out, start_next, p2, wait_pe):
            r4 = b
            if wait_out:
                b3 = b - 3
                v3, bb3 = (v, b3) if b3 >= 0 else (v - 1, b3 + nb)
                out_desc(v3, bb3, (b + 1) % 4).wait()
            if start_next:
                bn = b + 1
                vn, bbn = (v, bn) if bn < nb else (v + 1, 0)
                gather_desc(vn, bbn, (b + 1) % 4).start()
            gather_desc(v, b, r4).wait()
            if wait_pe:
                pe_desc(v, p2).wait()
            compute(r4, p2)
            out_desc(v, b, r4).start()

        def do_window(v, p2, first_win, last_win):
            if not last_win:
                pe_desc(v + 1, p2 ^ 1).start()
            for b in range(nb):
                wait_out = not (first_win and b < 3)
                start_next = not (last_win and b == nb - 1)
                chunk_step(v, b, wait_out, start_next, p2, b == 0)

        gather_desc(0, 0, 0).start()
        pe_desc(0, 0).start()
        do_window(0, 0, True, False)

        def pair_body(t, carry):
            v = 1 + 2 * t
            do_window(v, 1, False, False)
            do_window(v + 1, 0, False, False)
            return carry

        lax.fori_loop(0, (nwin - 2) // 2, pair_body, 0)

        do_window(nwin - 1, (nwin - 1) & 1, False, True)
        for b in range(1, nb):
            out_desc(nwin - 1, b, b).wait()

    return k(x_flat, table, pe)


def kernel(x, table):
    b, s = x.shape
    pe = jnp.asarray(_make_pe(s, _D))
    x_flat = x.reshape(-1).astype(jnp.int32)
    out = _sc_embed(x_flat, table, pe, b, s)
    return out.reshape(b, s, _D)
